# Initial kernel scaffold; baseline (speedup 1.0000x reference)
import jax
import jax.numpy as jnp
from jax.experimental import pallas as pl


def kernel(user_inputs, item_inputs, user_tables, item_tables, uW1, ub1, ug1, ube1, uW2, ub2, ug2, ube2, iW1, ib1, ig1, ibe1, iW2, ib2, ig2, ibe2):
    raise NotImplementedError("write your pallas kernel here")


# speedup vs baseline: 8.2993x; 8.2993x over previous
import functools

import jax
import jax.numpy as jnp
from jax import lax
from jax.experimental import pallas as pl
from jax.experimental.pallas import tpu as pltpu
from jax.experimental.pallas import tpu_sc as plsc

B = 16384
NF = 13
V = 100000
D = 32
DIN = NF * D
H1, H2 = 128, 64
EPS_BN = 1e-5
EPS_NORM = 1e-12

NC, NS = 2, 16
NW = NC * NS
RW = B // NW
IPW = RW * NF
IPG = 128
G = IPW // IPG
NBUF = 4
NGRP = G // NBUF

def _sc_gather_body(u_tab, i_tab, u_idx, i_idx, u_out, i_out,
                    idx_v, bufs, sem_g, sem_o):
    wid = lax.axis_index("s") * NC + lax.axis_index("c")
    for tab, idx_hbm, out in ((u_tab, u_idx, u_out), (i_tab, i_idx, i_out)):
        pltpu.sync_copy(idx_hbm.at[wid], idx_v)
        obase = wid * IPW

        def group(g, carry):
            gets = [
                pltpu.async_copy(tab.at[idx_v.at[g * NBUF + b]],
                                 bufs.at[b], sem_g)
                for b in range(NBUF)
            ]
            puts = []
            for b in range(NBUF):
                gets[b].wait()
                puts.append(pltpu.async_copy(
                    bufs.at[b],
                    out.at[pl.ds(obase + (g * NBUF + b) * IPG, IPG)],
                    sem_o))
            for put in puts:
                put.wait()
            return carry

        lax.fori_loop(0, NGRP, group, 0)


@functools.cache
def _sc_gather_call():
    mesh = plsc.VectorSubcoreMesh(core_axis_name="c", subcore_axis_name="s",
                                  num_cores=NC, num_subcores=NS)
    return pl.kernel(
        _sc_gather_body,
        mesh=mesh,
        out_type=[jax.ShapeDtypeStruct((B * NF, D), jnp.float32),
                  jax.ShapeDtypeStruct((B * NF, D), jnp.float32)],
        scratch_types=[pltpu.VMEM((G, IPG), jnp.int32),
                       pltpu.VMEM((NBUF, IPG, D), jnp.float32),
                       pltpu.SemaphoreType.DMA,
                       pltpu.SemaphoreType.DMA],
        compiler_params=pltpu.CompilerParams(use_tc_tiling_on_sc=False),
    )


BLK = 1024
NB = B // BLK


def _mlp_body(uf, itf, uW1r, iW1r, uW2r, iW2r, uv1, iv1, uv2, iv2, out,
              h1u, h1i, h2u, h2i, s1u, s1i, s2u, s2i):
    p = pl.program_id(0)
    i = pl.program_id(1)
    towers = ((uf, uW1r, uW2r, uv1, uv2, h1u, h2u, s1u, s2u),
              (itf, iW1r, iW2r, iv1, iv2, h1i, h2i, s1i, s2i))

    @pl.when(p == 0)
    def _():
        for f_ref, W1r, _W2r, _v1, _v2, h1, _h2, s1, _s2 in towers:
            x = f_ref[...]
            h = jnp.dot(x, W1r[...], preferred_element_type=jnp.float32)
            h1[pl.ds(i * BLK, BLK), :] = h
            st = jnp.concatenate(
                [jnp.sum(h, axis=0, keepdims=True),
                 jnp.sum(h * h, axis=0, keepdims=True)], axis=0)

            @pl.when(i == 0)
            def _():
                s1[0:2, :] = st

            @pl.when(i > 0)
            def _():
                s1[0:2, :] = s1[0:2, :] + st

    @pl.when(p == 1)
    def _():
        for _f, _W1r, W2r, v1, _v2, h1, h2, s1, s2 in towers:
            mu = s1[0:1, :] * (1.0 / B)
            var = s1[1:2, :] * (1.0 / B) - mu * mu
            a = v1[0:1, :] * lax.rsqrt(var + EPS_BN)
            c = v1[1:2, :] - mu * a
            h = h1[pl.ds(i * BLK, BLK), :]
            t = jnp.tanh(h * a + c)
            h2blk = jnp.dot(t, W2r[...], preferred_element_type=jnp.float32)
            h2[pl.ds(i * BLK, BLK), :] = h2blk
            st = jnp.concatenate(
                [jnp.sum(h2blk, axis=0, keepdims=True),
                 jnp.sum(h2blk * h2blk, axis=0, keepdims=True)], axis=0)

            @pl.when(i == 0)
            def _():
                s2[0:2, :] = st

            @pl.when(i > 0)
            def _():
                s2[0:2, :] = s2[0:2, :] + st

    @pl.when(p == 2)
    def _():
        zs = []
        for _f, _W1r, _W2r, _v1, v2, _h1, h2, _s1, s2 in towers:
            mu = s2[0:1, :] * (1.0 / B)
            var = s2[1:2, :] * (1.0 / B) - mu * mu
            a = v2[0:1, :] * lax.rsqrt(var + EPS_BN)
            c = v2[1:2, :] - mu * a
            zs.append(jnp.tanh(h2[pl.ds(i * BLK, BLK), :] * a + c))
        zu, zi = zs
        nu = jnp.maximum(jnp.sqrt(jnp.sum(zu * zu, axis=1, keepdims=True)),
                         EPS_NORM)
        ni = jnp.maximum(jnp.sqrt(jnp.sum(zi * zi, axis=1, keepdims=True)),
                         EPS_NORM)
        score = jnp.sum(zu * zi, axis=1, keepdims=True) / (nu * ni)
        out[pl.ds(i * BLK, BLK), :] = score


def _mlp_call(u_feat, i_feat, uW1, iW1, uW2, iW2, uv1, iv1, uv2, iv2):
    feat_spec = pl.BlockSpec((BLK, DIN),
                             lambda p, i: (jnp.where(p == 0, i, NB - 1), 0))
    whole = lambda shape: pl.BlockSpec(shape, lambda p, i: (0, 0))
    return pl.pallas_call(
        _mlp_body,
        grid=(3, NB),
        in_specs=[feat_spec, feat_spec,
                  whole((DIN, H1)), whole((DIN, H1)),
                  whole((H1, H2)), whole((H1, H2)),
                  whole((8, H1)), whole((8, H1)),
                  whole((8, H2)), whole((8, H2))],
        out_specs=pl.BlockSpec((B, 1), lambda p, i: (0, 0)),
        out_shape=jax.ShapeDtypeStruct((B, 1), jnp.float32),
        scratch_shapes=[pltpu.VMEM((B, H1), jnp.float32),
                        pltpu.VMEM((B, H1), jnp.float32),
                        pltpu.VMEM((B, H2), jnp.float32),
                        pltpu.VMEM((B, H2), jnp.float32),
                        pltpu.VMEM((8, H1), jnp.float32),
                        pltpu.VMEM((8, H1), jnp.float32),
                        pltpu.VMEM((8, H2), jnp.float32),
                        pltpu.VMEM((8, H2), jnp.float32)],
    )(u_feat, i_feat, uW1, iW1, uW2, iW2, uv1, iv1, uv2, iv2)


def _pack_bn(g, be):
    v = jnp.stack([g, be])
    return jnp.concatenate([v, jnp.zeros((6, v.shape[1]), jnp.float32)], axis=0)


def kernel(user_inputs, item_inputs, user_tables, item_tables,
           uW1, ub1, ug1, ube1, uW2, ub2, ug2, ube2,
           iW1, ib1, ig1, ibe1, iW2, ib2, ig2, ibe2):
    offs = (jnp.arange(NF, dtype=jnp.int32) * V)[None, :]
    u_idx = (user_inputs.astype(jnp.int32) + offs).reshape(NW, G, IPG)
    i_idx = (item_inputs.astype(jnp.int32) + offs).reshape(NW, G, IPG)
    u_tab = user_tables.reshape(NF * V, D)
    i_tab = item_tables.reshape(NF * V, D)
    u_flat, i_flat = _sc_gather_call()(u_tab, i_tab, u_idx, i_idx)
    u_feat = u_flat.reshape(B, DIN)
    i_feat = i_flat.reshape(B, DIN)
    score = _mlp_call(u_feat, i_feat, uW1, iW1, uW2, iW2,
                      _pack_bn(ug1, ube1), _pack_bn(ig1, ibe1),
                      _pack_bn(ug2, ube2), _pack_bn(ig2, ibe2))
    return score.reshape(B)


# speedup vs baseline: 11.9156x; 1.4357x over previous
import functools

import jax
import jax.numpy as jnp
from jax import lax
from jax.experimental import pallas as pl
from jax.experimental.pallas import tpu as pltpu
from jax.experimental.pallas import tpu_sc as plsc

B = 16384
NF = 13
V = 100000
Vp = 102400
D = 32
DIN = NF * D
H1, H2 = 128, 64
EPS_BN = 1e-5
EPS_NORM = 1e-12

VB = 4096
NVB = Vp // VB
PR = NF * Vp


def _pack_body(u_ref, i_ref, uo_ref, io_ref):
    for src, dst in ((u_ref, uo_ref), (i_ref, io_ref)):
        x = src[0]
        dst[:, 0:D] = jnp.swapaxes(x, 0, 1)


def _pack_call(tabT_u, tabT_i):
    in_spec = pl.BlockSpec((1, D, VB), lambda f, vc: (f, 0, vc))
    out_spec = pl.BlockSpec((VB, 128), lambda f, vc: (f * NVB + vc, 0))
    return pl.pallas_call(
        _pack_body,
        grid=(NF, NVB),
        in_specs=[in_spec, in_spec],
        out_specs=[out_spec, out_spec],
        out_shape=[jax.ShapeDtypeStruct((PR, 128), jnp.float32),
                   jax.ShapeDtypeStruct((PR, 128), jnp.float32)],
    )(tabT_u, tabT_i)


NC, NS = 2, 16
NW = NC * NS
RW = B // NW
IPW = RW * NF
IPG = 128
G = IPW // IPG
NBUF = 4
NGRP = G // NBUF


def _sc_gather_body(u_tab, i_tab, u_rows, i_rows, u_out, i_out,
                    rows_v, pad_bufs, sem_g, sem_o):
    wid = lax.axis_index("s") * NC + lax.axis_index("c")
    for tab, rows_hbm, out in ((u_tab, u_rows, u_out),
                               (i_tab, i_rows, i_out)):
        pltpu.sync_copy(rows_hbm.at[wid], rows_v)
        obase = wid * IPW

        def group(g, carry):
            gets = [
                pltpu.async_copy(tab.at[rows_v.at[g * NBUF + b]],
                                 pad_bufs.at[b], sem_g)
                for b in range(NBUF)
            ]
            puts = []
            for b in range(NBUF):
                gets[b].wait()
                puts.append(pltpu.async_copy(
                    pad_bufs.at[b, :, pl.ds(0, D)],
                    out.at[pl.ds(obase + (g * NBUF + b) * IPG, IPG)],
                    sem_o))
            for put in puts:
                put.wait()
            return carry

        lax.fori_loop(0, NGRP, group, 0)


@functools.cache
def _sc_gather_call():
    mesh = plsc.VectorSubcoreMesh(core_axis_name="c", subcore_axis_name="s",
                                  num_cores=NC, num_subcores=NS)
    return pl.kernel(
        _sc_gather_body,
        mesh=mesh,
        out_type=[jax.ShapeDtypeStruct((B * NF, D), jnp.float32),
                  jax.ShapeDtypeStruct((B * NF, D), jnp.float32)],
        scratch_types=[pltpu.VMEM((G, IPG), jnp.int32),
                       pltpu.VMEM((NBUF, IPG, 128), jnp.float32),
                       pltpu.SemaphoreType.DMA,
                       pltpu.SemaphoreType.DMA],
        compiler_params=pltpu.CompilerParams(use_tc_tiling_on_sc=False,
                                             needs_layout_passes=False),
    )


BLK = 1024
NB = B // BLK


def _mlp_body(uf, itf, uW1r, iW1r, uW2r, iW2r, uv1, iv1, uv2, iv2, out,
              h1u, h1i, h2u, h2i, s1u, s1i, s2u, s2i):
    p = pl.program_id(0)
    i = pl.program_id(1)
    towers = ((uf, uW1r, uW2r, uv1, uv2, h1u, h2u, s1u, s2u),
              (itf, iW1r, iW2r, iv1, iv2, h1i, h2i, s1i, s2i))

    @pl.when(p == 0)
    def _():
        for f_ref, W1r, _W2r, _v1, _v2, h1, _h2, s1, _s2 in towers:
            x = f_ref[...]
            h = jnp.dot(x, W1r[...], preferred_element_type=jnp.float32)
            h1[pl.ds(i * BLK, BLK), :] = h
            st = jnp.concatenate(
                [jnp.sum(h, axis=0, keepdims=True),
                 jnp.sum(h * h, axis=0, keepdims=True)], axis=0)

            @pl.when(i == 0)
            def _():
                s1[0:2, :] = st

            @pl.when(i > 0)
            def _():
                s1[0:2, :] = s1[0:2, :] + st

    @pl.when(p == 1)
    def _():
        for _f, _W1r, W2r, v1, _v2, h1, h2, s1, s2 in towers:
            mu = s1[0:1, :] * (1.0 / B)
            var = s1[1:2, :] * (1.0 / B) - mu * mu
            a = v1[0:1, :] * lax.rsqrt(var + EPS_BN)
            c = v1[1:2, :] - mu * a
            h = h1[pl.ds(i * BLK, BLK), :]
            t = jnp.tanh(h * a + c)
            h2blk = jnp.dot(t, W2r[...], preferred_element_type=jnp.float32)
            h2[pl.ds(i * BLK, BLK), :] = h2blk
            st = jnp.concatenate(
                [jnp.sum(h2blk, axis=0, keepdims=True),
                 jnp.sum(h2blk * h2blk, axis=0, keepdims=True)], axis=0)

            @pl.when(i == 0)
            def _():
                s2[0:2, :] = st

            @pl.when(i > 0)
            def _():
                s2[0:2, :] = s2[0:2, :] + st

    @pl.when(p == 2)
    def _():
        zs = []
        for _f, _W1r, _W2r, _v1, v2, _h1, h2, _s1, s2 in towers:
            mu = s2[0:1, :] * (1.0 / B)
            var = s2[1:2, :] * (1.0 / B) - mu * mu
            a = v2[0:1, :] * lax.rsqrt(var + EPS_BN)
            c = v2[1:2, :] - mu * a
            zs.append(jnp.tanh(h2[pl.ds(i * BLK, BLK), :] * a + c))
        zu, zi = zs
        nu = jnp.maximum(jnp.sqrt(jnp.sum(zu * zu, axis=1, keepdims=True)),
                         EPS_NORM)
        ni = jnp.maximum(jnp.sqrt(jnp.sum(zi * zi, axis=1, keepdims=True)),
                         EPS_NORM)
        score = jnp.sum(zu * zi, axis=1, keepdims=True) / (nu * ni)
        out[pl.ds(i * BLK, BLK), :] = score


def _mlp_call(u_feat, i_feat, uW1, iW1, uW2, iW2, uv1, iv1, uv2, iv2):
    feat_spec = pl.BlockSpec((BLK, DIN),
                             lambda p, i: (jnp.where(p == 0, i, NB - 1), 0))
    whole = lambda shape: pl.BlockSpec(shape, lambda p, i: (0, 0))
    return pl.pallas_call(
        _mlp_body,
        grid=(3, NB),
        in_specs=[feat_spec, feat_spec,
                  whole((DIN, H1)), whole((DIN, H1)),
                  whole((H1, H2)), whole((H1, H2)),
                  whole((8, H1)), whole((8, H1)),
                  whole((8, H2)), whole((8, H2))],
        out_specs=pl.BlockSpec((B, 1), lambda p, i: (0, 0)),
        out_shape=jax.ShapeDtypeStruct((B, 1), jnp.float32),
        scratch_shapes=[pltpu.VMEM((B, H1), jnp.float32),
                        pltpu.VMEM((B, H1), jnp.float32),
                        pltpu.VMEM((B, H2), jnp.float32),
                        pltpu.VMEM((B, H2), jnp.float32),
                        pltpu.VMEM((8, H1), jnp.float32),
                        pltpu.VMEM((8, H1), jnp.float32),
                        pltpu.VMEM((8, H2), jnp.float32),
                        pltpu.VMEM((8, H2), jnp.float32)],
    )(u_feat, i_feat, uW1, iW1, uW2, iW2, uv1, iv1, uv2, iv2)


def _pack_bn(g, be):
    v = jnp.stack([g, be])
    return jnp.concatenate([v, jnp.zeros((6, v.shape[1]), jnp.float32)], axis=0)


def kernel(user_inputs, item_inputs, user_tables, item_tables,
           uW1, ub1, ug1, ube1, uW2, ub2, ug2, ube2,
           iW1, ib1, ig1, ibe1, iW2, ib2, ig2, ibe2):
    offs = (jnp.arange(NF, dtype=jnp.int32) * Vp)[None, :]
    u_rows = (user_inputs.astype(jnp.int32) + offs).reshape(NW, G, IPG)
    i_rows = (item_inputs.astype(jnp.int32) + offs).reshape(NW, G, IPG)
    tabT_u = jnp.swapaxes(user_tables, 1, 2)
    tabT_i = jnp.swapaxes(item_tables, 1, 2)
    pu, pi = _pack_call(tabT_u, tabT_i)
    u_flat, i_flat = _sc_gather_call()(pu, pi, u_rows, i_rows)
    u_feat = u_flat.reshape(B, DIN)
    i_feat = i_flat.reshape(B, DIN)
    score = _mlp_call(u_feat, i_feat, uW1, iW1, uW2, iW2,
                      _pack_bn(ug1, ube1), _pack_bn(ig1, ibe1),
                      _pack_bn(ug2, ube2), _pack_bn(ig2, ibe2))
    return score.reshape(B)
